# baseline (speedup 1.0000x reference)
import jax
import jax.numpy as jnp
from jax import lax
from jax.experimental import pallas as pl
from jax.experimental.pallas import tpu as pltpu
from jax.experimental.pallas import tpu_sc as plsc

_RATE = 0.3
_B, _S, _F = 16384, 20, 64
_NCAT, _NNUM = 8, 16
_BBLK = 512

_NW = 32
_ROWS = _NCAT * _B
_RPW = _ROWS // _NW
_CH = 128
_NCHUNK = _RPW // _CH
_CPR = 4
_NROUND = _NCHUNK // _CPR
_TW = 32


def _sc_gather_body(tab_ref, idx_ref, out_ref, idx_v, rows_v, sem):
    wid = lax.axis_index("s") * 2 + lax.axis_index("c")
    base = wid * _RPW
    pltpu.sync_copy(idx_ref.at[wid], idx_v)
    for r in range(_NROUND):
        descs = [
            pltpu.async_copy(tab_ref.at[idx_v.at[r * _CPR + k]],
                             rows_v.at[pl.ds(k * _CH, _CH)], sem)
            for k in range(_CPR)
        ]
        for d in descs:
            d.wait()
        pltpu.sync_copy(rows_v,
                        out_ref.at[pl.ds(base + r * _CPR * _CH, _CPR * _CH)])


def _gather_rows(table, idx):
    mesh = plsc.VectorSubcoreMesh(core_axis_name="c", subcore_axis_name="s")
    return pl.kernel(
        _sc_gather_body,
        out_type=jax.ShapeDtypeStruct((_ROWS, _TW), jnp.float32),
        mesh=mesh,
        scratch_types=[
            pltpu.VMEM((_NCHUNK, _CH), jnp.int32),
            pltpu.VMEM((_CPR * _CH, _TW), jnp.float32),
            pltpu.SemaphoreType.DMA,
        ],
        compiler_params=pltpu.CompilerParams(use_tc_tiling_on_sc=False),
    )(table, idx.reshape(_NW, _NCHUNK, _CH))


def _prep_body(x_ref, xt_ref, o_ref):
    g = pl.program_id(0)
    xb = x_ref[...]
    xr = xb.reshape(_BBLK, _S, _F)
    t = jnp.transpose(xr[:, :, :_NCAT], (0, 2, 1))
    t = jnp.concatenate(
        [t, jnp.zeros((_BBLK, _NCAT, _TW - _S), jnp.float32)], axis=-1)
    xt_ref[...] = t.reshape(_BBLK, _NCAT * _TW)
    xn = xr[:, :, _NCAT:_NCAT + _NNUM]
    s1 = jnp.sum(xn, axis=(0, 1))
    s2 = jnp.sum(xn * xn, axis=(0, 1))
    part = jnp.stack([s1, s2])

    @pl.when(g == 0)
    def _init():
        o_ref[...] = part

    @pl.when(g > 0)
    def _acc():
        o_ref[...] = o_ref[...] + part


_KM1, _KM2 = 0x6D3E048F, 0x1022172D


def _threefry_mask(flat_idx):
    rots = ((13, 15, 26, 6), (17, 29, 16, 24))
    ks = (jnp.uint32(_KM1), jnp.uint32(_KM2),
          jnp.uint32(_KM1 ^ _KM2 ^ 0x1BD11BDA))
    x0 = jnp.zeros_like(flat_idx) + ks[0]
    x1 = flat_idx + ks[1]
    for rr, a, b, c in ((0, 1, 2, 1), (1, 2, 0, 2), (0, 0, 1, 3),
                        (1, 1, 2, 4), (0, 2, 0, 5)):
        for r in rots[rr]:
            x0 = x0 + x1
            x1 = (x1 << r) | (x1 >> (32 - r))
            x1 = x0 ^ x1
        x0 = x0 + ks[a]
        x1 = x1 + ks[b] + jnp.uint32(c)
    bits = x0 ^ x1
    return lax.bitcast_convert_type(
        (bits >> 9) | jnp.uint32(0x3F800000), jnp.float32) - 1.0


def _merge_body(x_ref, shuf_ref, noise_ref, sums_ref, o_ref, m_ref):
    g = pl.program_id(0)
    xb = x_ref[...]
    xr = xb.reshape(_BBLK, _S, _F)
    row = lax.broadcasted_iota(jnp.uint32, (_BBLK, _S * _F), 0)
    lane = lax.broadcasted_iota(jnp.uint32, (_BBLK, _S * _F), 1)
    flat = (jnp.uint32(g) * _BBLK + row) * (_S * _F) + lane
    u = _threefry_mask(flat)
    m_ref[...] = (u < _RATE).astype(jnp.int8)
    m = u.reshape(_BBLK, _S, _F) < _RATE
    sums = sums_ref[...]
    n = jnp.float32(_B * _S)
    var = (sums[1] - sums[0] * sums[0] / n) / (n - 1.0)
    std = jnp.sqrt(var)
    sb = shuf_ref[...].reshape(_BBLK, _NCAT, _TW)
    cat = jnp.transpose(sb, (0, 2, 1))[:, :_S, :]
    nb = noise_ref[...].reshape(_NNUM, _BBLK, _S)
    nz = jnp.transpose(nb, (1, 2, 0))
    repl = jnp.concatenate(
        [cat, nz * std[None, None, :], xr[:, :, _NCAT + _NNUM:]], axis=-1)
    o_ref[...] = jnp.where(m, repl, xr).reshape(_BBLK, _S * _F)


def kernel(x):
    k = jax.random.key(42)
    _, k_rest = jax.random.split(k)

    perms = jnp.stack(
        [jax.random.permutation(jax.random.fold_in(k_rest, i), _B)
         for i in range(_NCAT)])
    idx_flat = (perms.T.astype(jnp.int32) * _NCAT
                + jnp.arange(_NCAT, dtype=jnp.int32)[None, :]).reshape(-1)

    nkeys = jax.vmap(lambda j: jax.random.fold_in(k_rest, j))(
        jnp.arange(1000, 1000 + _NNUM))
    noiseT = jax.vmap(
        lambda kk: jax.random.normal(kk, (_B * _S,), jnp.float32))(nkeys)

    x2 = x.reshape(_B, _S * _F)
    xT, sums = pl.pallas_call(
        _prep_body,
        grid=(_B // _BBLK,),
        in_specs=[pl.BlockSpec((_BBLK, _S * _F), lambda g: (g, 0))],
        out_specs=[
            pl.BlockSpec((_BBLK, _NCAT * _TW), lambda g: (g, 0)),
            pl.BlockSpec((2, _NNUM), lambda g: (0, 0)),
        ],
        out_shape=[
            jax.ShapeDtypeStruct((_B, _NCAT * _TW), jnp.float32),
            jax.ShapeDtypeStruct((2, _NNUM), jnp.float32),
        ],
    )(x2)
    shufT = _gather_rows(xT.reshape(_ROWS, _TW), idx_flat)

    corrupted, mask8 = pl.pallas_call(
        _merge_body,
        grid=(_B // _BBLK,),
        in_specs=[
            pl.BlockSpec((_BBLK, _S * _F), lambda g: (g, 0)),
            pl.BlockSpec((_BBLK, _NCAT * _TW), lambda g: (g, 0)),
            pl.BlockSpec((_NNUM, _BBLK * _S), lambda g: (0, g)),
            pl.BlockSpec((2, _NNUM), lambda g: (0, 0)),
        ],
        out_specs=[
            pl.BlockSpec((_BBLK, _S * _F), lambda g: (g, 0)),
            pl.BlockSpec((_BBLK, _S * _F), lambda g: (g, 0)),
        ],
        out_shape=[
            jax.ShapeDtypeStruct((_B, _S * _F), jnp.float32),
            jax.ShapeDtypeStruct((_B, _S * _F), jnp.int8),
        ],
    )(x2, shufT.reshape(_B, _NCAT * _TW), noiseT, sums)

    mask = mask8.reshape(_B, _S, _F).astype(jnp.bool_)
    return corrupted.reshape(_B, _S, _F), mask, x


# speedup vs baseline: 1.0035x; 1.0035x over previous
import jax
import jax.numpy as jnp
from jax import lax
from jax.experimental import pallas as pl
from jax.experimental.pallas import tpu as pltpu
from jax.experimental.pallas import tpu_sc as plsc

_RATE = 0.3
_B, _S, _F = 16384, 20, 64
_NCAT, _NNUM = 8, 16
_BBLK = 256

_NW = 32
_ROWS = _NCAT * _B
_RPW = _ROWS // _NW
_CH = 128
_NCHUNK = _RPW // _CH
_CPR = 4
_NROUND = _NCHUNK // _CPR
_TW = 32


def _sc_gather_body(tab_ref, idx_ref, out_ref, idx_v, rows_v, sem):
    wid = lax.axis_index("s") * 2 + lax.axis_index("c")
    base = wid * _RPW
    pltpu.sync_copy(idx_ref.at[wid], idx_v)
    for r in range(_NROUND):
        descs = [
            pltpu.async_copy(tab_ref.at[idx_v.at[r * _CPR + k]],
                             rows_v.at[pl.ds(k * _CH, _CH)], sem)
            for k in range(_CPR)
        ]
        for d in descs:
            d.wait()
        pltpu.sync_copy(rows_v,
                        out_ref.at[pl.ds(base + r * _CPR * _CH, _CPR * _CH)])


def _gather_rows(table, idx):
    mesh = plsc.VectorSubcoreMesh(core_axis_name="c", subcore_axis_name="s")
    return pl.kernel(
        _sc_gather_body,
        out_type=jax.ShapeDtypeStruct((_ROWS, _TW), jnp.float32),
        mesh=mesh,
        scratch_types=[
            pltpu.VMEM((_NCHUNK, _CH), jnp.int32),
            pltpu.VMEM((_CPR * _CH, _TW), jnp.float32),
            pltpu.SemaphoreType.DMA,
        ],
        compiler_params=pltpu.CompilerParams(use_tc_tiling_on_sc=False),
    )(table, idx.reshape(_NW, _NCHUNK, _CH))


def _prep_body(x_ref, xt_ref, o_ref):
    g = pl.program_id(0)
    xb = x_ref[...]
    xr = xb.reshape(_BBLK, _S, _F)
    t = jnp.transpose(xr[:, :, :_NCAT], (0, 2, 1))
    t = jnp.concatenate(
        [t, jnp.zeros((_BBLK, _NCAT, _TW - _S), jnp.float32)], axis=-1)
    xt_ref[...] = t.reshape(_BBLK, _NCAT * _TW)
    xn = xr[:, :, _NCAT:_NCAT + _NNUM]
    s1 = jnp.sum(xn, axis=(0, 1))
    s2 = jnp.sum(xn * xn, axis=(0, 1))
    part = jnp.stack([s1, s2])

    @pl.when(g == 0)
    def _init():
        o_ref[...] = part

    @pl.when(g > 0)
    def _acc():
        o_ref[...] = o_ref[...] + part


_KM1, _KM2 = 0x6D3E048F, 0x1022172D


def _threefry_mask(flat_idx):
    rots = ((13, 15, 26, 6), (17, 29, 16, 24))
    ks = (jnp.uint32(_KM1), jnp.uint32(_KM2),
          jnp.uint32(_KM1 ^ _KM2 ^ 0x1BD11BDA))
    x0 = jnp.zeros_like(flat_idx) + ks[0]
    x1 = flat_idx + ks[1]
    for rr, a, b, c in ((0, 1, 2, 1), (1, 2, 0, 2), (0, 0, 1, 3),
                        (1, 1, 2, 4), (0, 2, 0, 5)):
        for r in rots[rr]:
            x0 = x0 + x1
            x1 = (x1 << r) | (x1 >> (32 - r))
            x1 = x0 ^ x1
        x0 = x0 + ks[a]
        x1 = x1 + ks[b] + jnp.uint32(c)
    bits = x0 ^ x1
    return lax.bitcast_convert_type(
        (bits >> 9) | jnp.uint32(0x3F800000), jnp.float32) - 1.0


def _merge_body(x_ref, shuf_ref, noise_ref, sums_ref, o_ref, m_ref):
    g = pl.program_id(0)
    xb = x_ref[...]
    xr = xb.reshape(_BBLK, _S, _F)
    row = lax.broadcasted_iota(jnp.uint32, (_BBLK, _S * _F), 0)
    lane = lax.broadcasted_iota(jnp.uint32, (_BBLK, _S * _F), 1)
    flat = (jnp.uint32(g) * _BBLK + row) * (_S * _F) + lane
    u = _threefry_mask(flat)
    m_ref[...] = (u < _RATE).astype(jnp.int8)
    m = u.reshape(_BBLK, _S, _F) < _RATE
    sums = sums_ref[...]
    n = jnp.float32(_B * _S)
    var = (sums[1] - sums[0] * sums[0] / n) / (n - 1.0)
    std = jnp.sqrt(var)
    sb = shuf_ref[...].reshape(_BBLK, _NCAT, _TW)
    cat = jnp.transpose(sb, (0, 2, 1))[:, :_S, :]
    nb = noise_ref[...].reshape(_NNUM, _BBLK, _S)
    nz = jnp.transpose(nb, (1, 2, 0))
    repl = jnp.concatenate(
        [cat, nz * std[None, None, :], xr[:, :, _NCAT + _NNUM:]], axis=-1)
    o_ref[...] = jnp.where(m, repl, xr).reshape(_BBLK, _S * _F)


def kernel(x):
    k = jax.random.key(42)
    _, k_rest = jax.random.split(k)

    perms = jnp.stack(
        [jax.random.permutation(jax.random.fold_in(k_rest, i), _B)
         for i in range(_NCAT)])
    idx_flat = (perms.T.astype(jnp.int32) * _NCAT
                + jnp.arange(_NCAT, dtype=jnp.int32)[None, :]).reshape(-1)

    nkeys = jax.vmap(lambda j: jax.random.fold_in(k_rest, j))(
        jnp.arange(1000, 1000 + _NNUM))
    noiseT = jax.vmap(
        lambda kk: jax.random.normal(kk, (_B * _S,), jnp.float32))(nkeys)

    x2 = x.reshape(_B, _S * _F)
    xT, sums = pl.pallas_call(
        _prep_body,
        grid=(_B // _BBLK,),
        in_specs=[pl.BlockSpec((_BBLK, _S * _F), lambda g: (g, 0))],
        out_specs=[
            pl.BlockSpec((_BBLK, _NCAT * _TW), lambda g: (g, 0)),
            pl.BlockSpec((2, _NNUM), lambda g: (0, 0)),
        ],
        out_shape=[
            jax.ShapeDtypeStruct((_B, _NCAT * _TW), jnp.float32),
            jax.ShapeDtypeStruct((2, _NNUM), jnp.float32),
        ],
    )(x2)
    shufT = _gather_rows(xT.reshape(_ROWS, _TW), idx_flat)

    corrupted, mask8 = pl.pallas_call(
        _merge_body,
        grid=(_B // _BBLK,),
        in_specs=[
            pl.BlockSpec((_BBLK, _S * _F), lambda g: (g, 0)),
            pl.BlockSpec((_BBLK, _NCAT * _TW), lambda g: (g, 0)),
            pl.BlockSpec((_NNUM, _BBLK * _S), lambda g: (0, g)),
            pl.BlockSpec((2, _NNUM), lambda g: (0, 0)),
        ],
        out_specs=[
            pl.BlockSpec((_BBLK, _S * _F), lambda g: (g, 0)),
            pl.BlockSpec((_BBLK, _S * _F), lambda g: (g, 0)),
        ],
        out_shape=[
            jax.ShapeDtypeStruct((_B, _S * _F), jnp.float32),
            jax.ShapeDtypeStruct((_B, _S * _F), jnp.int8),
        ],
    )(x2, shufT.reshape(_B, _NCAT * _TW), noiseT, sums)

    mask = mask8.reshape(_B, _S, _F).astype(jnp.bool_)
    return corrupted.reshape(_B, _S, _F), mask, x
